# baseline (speedup 1.0000x reference)
import jax
import jax.numpy as jnp
from jax.experimental import pallas as pl


def _relu_body(x_ref, o_ref):
    o_ref[...] = jnp.maximum(x_ref[...], 0.0)


def kernel(inputs):
    shape = inputs.shape
    b0, b1, h, w, c = shape
    wc = w * c
    x = inputs.reshape(b0, b1, h, wc // 128, 128)
    block_h = 112
    spec = pl.BlockSpec(
        (1, 1, block_h, wc // 128, 128), lambda i, j, k: (i, j, k, 0, 0)
    )
    out = pl.pallas_call(
        _relu_body,
        grid=(b0, b1, h // block_h),
        in_specs=[spec],
        out_specs=spec,
        out_shape=jax.ShapeDtypeStruct(x.shape, jnp.float32),
    )(x)
    return out.reshape(shape)


# speedup vs baseline: 1.4816x; 1.4816x over previous
import jax
import jax.numpy as jnp
from jax.experimental import pallas as pl


def _relu_body(x_ref, o_ref):
    o_ref[...] = jnp.maximum(x_ref[...], 0.0)


def kernel(inputs):
    shape = inputs.shape
    n = inputs.size
    rows = n // 128
    x = inputs.reshape(rows, 128)
    block_rows = 10752
    spec = pl.BlockSpec((block_rows, 128), lambda i: (i, 0))
    out = pl.pallas_call(
        _relu_body,
        grid=(rows // block_rows,),
        in_specs=[spec],
        out_specs=spec,
        out_shape=jax.ShapeDtypeStruct(x.shape, jnp.float32),
    )(x)
    return out.reshape(shape)


# speedup vs baseline: 18.1479x; 12.2488x over previous
import jax
import jax.numpy as jnp
from jax.experimental import pallas as pl


def _relu_body(x_ref, o_ref):
    o_ref[...] = jnp.maximum(x_ref[...], 0.0)


def kernel(inputs):
    b0, b1, h, w, c = inputs.shape
    xt = inputs.transpose(0, 1, 2, 4, 3)
    block_h = 56
    spec = pl.BlockSpec(
        (1, 1, block_h, c, w), lambda i, j, k: (i, j, k, 0, 0)
    )
    out = pl.pallas_call(
        _relu_body,
        grid=(b0, b1, h // block_h),
        in_specs=[spec],
        out_specs=spec,
        out_shape=jax.ShapeDtypeStruct(xt.shape, jnp.float32),
    )(xt)
    return out.transpose(0, 1, 2, 4, 3)


# speedup vs baseline: 18.3001x; 1.0084x over previous
import jax
import jax.numpy as jnp
from jax.experimental import pallas as pl


def _relu_body(x_ref, o_ref):
    o_ref[...] = jnp.maximum(x_ref[...], 0.0)


def kernel(inputs):
    b0, b1, h, w, c = inputs.shape
    xt = inputs.transpose(0, 1, 2, 4, 3)
    block_h = 112
    spec = pl.BlockSpec(
        (1, 1, block_h, c, w), lambda i, j, k: (i, j, k, 0, 0)
    )
    out = pl.pallas_call(
        _relu_body,
        grid=(b0, b1, h // block_h),
        in_specs=[spec],
        out_specs=spec,
        out_shape=jax.ShapeDtypeStruct(xt.shape, jnp.float32),
    )(xt)
    return out.transpose(0, 1, 2, 4, 3)
